# baseline (speedup 1.0000x reference)
import functools

import jax
import jax.numpy as jnp
from jax import lax
from jax.experimental import pallas as pl
from jax.experimental.pallas import tpu as pltpu

ALPHA = 0.2
N = 2048
F = 16
ROW_BLK = 256


def _fused_gat_body(x_ref, w1_ref, a1_ref, w2_ref, a2_ref, o_ref, wh_s, h_s):
    xb = x_ref[0]

    def layer(xin, W, a, write_out):
        wh = jnp.dot(xin, W, preferred_element_type=jnp.float32)
        wh_s[...] = wh
        f1 = jnp.dot(wh, a[:F, :], preferred_element_type=jnp.float32)
        f2c = jnp.dot(wh, a[F:, :], preferred_element_type=jnp.float32)
        f2r = lax.dot_general(
            a[F:, :], wh,
            dimension_numbers=(((0,), (1,)), ((), ())),
            preferred_element_type=jnp.float32,
        )
        u = jnp.exp(f1)
        v = jnp.exp(ALPHA * f1)
        g = jnp.exp(f2c)
        gh = jnp.exp(ALPHA * f2c)
        ones = jnp.ones((N, 1), jnp.float32)
        qp = (jnp.concatenate([wh, ones], axis=1) * g).astype(jnp.bfloat16)
        qn = (jnp.concatenate([wh, ones], axis=1) * gh).astype(jnp.bfloat16)
        tn = jnp.sum(qn.astype(jnp.float32), axis=0, keepdims=True)
        for j in range(N // ROW_BLK):
            sl = slice(j * ROW_BLK, (j + 1) * ROW_BLK)
            mask = jnp.where(f2r >= -f1[sl, :], 1.0, 0.0).astype(jnp.bfloat16)
            mp = jnp.dot(mask, qp, preferred_element_type=jnp.float32)
            mn = jnp.dot(mask, qn, preferred_element_type=jnp.float32)
            mn = tn - mn
            num = u[sl, :] * mp[:, :F] + v[sl, :] * mn[:, :F]
            den = u[sl, :] * mp[:, F:] + v[sl, :] * mn[:, F:]
            h = num / den
            write_out(j, jnp.where(h > 0, h, jnp.exp(h) - 1.0))

    def write_h(j, val):
        h_s[pl.ds(j * ROW_BLK, ROW_BLK), :] = val

    def write_o(j, val):
        o_ref[0, pl.ds(j * ROW_BLK, ROW_BLK), :] = val

    layer(xb, w1_ref[...], a1_ref[...], write_h)
    layer(h_s[...], w2_ref[...], a2_ref[...], write_o)


@jax.jit
def kernel(x, W1, a1, W2, a2):
    B = x.shape[0]
    grid = (B,)
    return pl.pallas_call(
        _fused_gat_body,
        grid=grid,
        in_specs=[
            pl.BlockSpec((1, N, F), lambda b: (b, 0, 0)),
            pl.BlockSpec((F, F), lambda b: (0, 0)),
            pl.BlockSpec((2 * F, 1), lambda b: (0, 0)),
            pl.BlockSpec((F, F), lambda b: (0, 0)),
            pl.BlockSpec((2 * F, 1), lambda b: (0, 0)),
        ],
        out_specs=pl.BlockSpec((1, N, F), lambda b: (b, 0, 0)),
        out_shape=jax.ShapeDtypeStruct((B, N, F), jnp.float32),
        scratch_shapes=[
            pltpu.VMEM((N, F), jnp.float32),
            pltpu.VMEM((N, F), jnp.float32),
        ],
    )(x, W1, a1, W2, a2)


# speedup vs baseline: 1.2123x; 1.2123x over previous
import functools

import jax
import jax.numpy as jnp
from jax import lax
from jax.experimental import pallas as pl
from jax.experimental.pallas import tpu as pltpu

ALPHA = 0.2
N = 2048
F = 16
ROW_BLK = 256


def _fused_gat_body(x_ref, w1_ref, a1_ref, w2_ref, a2_ref, o_ref, wh_s, h_s):
    xb = x_ref[0]

    def layer(xin, W, a, write_out):
        wh = jnp.dot(xin, W, preferred_element_type=jnp.float32)
        wh_s[...] = wh
        f1 = jnp.dot(wh, a[:F, :], preferred_element_type=jnp.float32)
        f2c = jnp.dot(wh, a[F:, :], preferred_element_type=jnp.float32)
        f2r = lax.dot_general(
            a[F:, :], wh,
            dimension_numbers=(((0,), (1,)), ((), ())),
            preferred_element_type=jnp.float32,
        )
        u = jnp.exp(f1)
        v = jnp.exp(ALPHA * f1)
        g = jnp.exp(f2c)
        gh = jnp.exp(ALPHA * f2c)
        ones = jnp.ones((N, 1), jnp.float32)
        who = jnp.concatenate([wh, ones], axis=1)
        q = jnp.concatenate([who * g, who * gh], axis=1)
        tn = jnp.sum(q[:, F + 1:], axis=0, keepdims=True)
        for j in range(N // ROW_BLK):
            sl = slice(j * ROW_BLK, (j + 1) * ROW_BLK)
            mask = jnp.where(f2r >= -f1[sl, :], 1.0, 0.0)
            m = jnp.dot(mask, q, preferred_element_type=jnp.float32)
            mp = m[:, :F + 1]
            mn = tn - m[:, F + 1:]
            num = u[sl, :] * mp[:, :F] + v[sl, :] * mn[:, :F]
            den = u[sl, :] * mp[:, F:] + v[sl, :] * mn[:, F:]
            h = num / den
            write_out(j, jnp.where(h > 0, h, jnp.exp(h) - 1.0))

    def write_h(j, val):
        h_s[pl.ds(j * ROW_BLK, ROW_BLK), :] = val

    def write_o(j, val):
        o_ref[0, pl.ds(j * ROW_BLK, ROW_BLK), :] = val

    layer(xb, w1_ref[...], a1_ref[...], write_h)
    layer(h_s[...], w2_ref[...], a2_ref[...], write_o)


@jax.jit
def kernel(x, W1, a1, W2, a2):
    B = x.shape[0]
    grid = (B,)
    return pl.pallas_call(
        _fused_gat_body,
        grid=grid,
        in_specs=[
            pl.BlockSpec((1, N, F), lambda b: (b, 0, 0)),
            pl.BlockSpec((F, F), lambda b: (0, 0)),
            pl.BlockSpec((2 * F, 1), lambda b: (0, 0)),
            pl.BlockSpec((F, F), lambda b: (0, 0)),
            pl.BlockSpec((2 * F, 1), lambda b: (0, 0)),
        ],
        out_specs=pl.BlockSpec((1, N, F), lambda b: (b, 0, 0)),
        out_shape=jax.ShapeDtypeStruct((B, N, F), jnp.float32),
        scratch_shapes=[
            pltpu.VMEM((N, F), jnp.float32),
            pltpu.VMEM((N, F), jnp.float32),
        ],
    )(x, W1, a1, W2, a2)


# speedup vs baseline: 1.2773x; 1.0536x over previous
import functools

import jax
import jax.numpy as jnp
from jax import lax
from jax.experimental import pallas as pl
from jax.experimental.pallas import tpu as pltpu

ALPHA = 0.2
N = 2048
F = 16
ROW_BLK = 256


def _fused_gat_body(x_ref, w1_ref, a1_ref, w2_ref, a2_ref, o_ref, wh_s, h_s):
    def layer(xin, W, a, write_out):
        wh = jnp.dot(xin, W, preferred_element_type=jnp.float32)
        wh_s[...] = wh
        f1 = jnp.dot(wh, a[:F, :], preferred_element_type=jnp.float32)
        f2c = jnp.dot(wh, a[F:, :], preferred_element_type=jnp.float32)
        f2r = lax.dot_general(
            a[F:, :], wh,
            dimension_numbers=(((0,), (1,)), ((), ())),
            preferred_element_type=jnp.float32,
        )
        u = jnp.exp(f1)
        v = jnp.exp(ALPHA * f1)
        g = jnp.exp(f2c)
        gh = jnp.exp(ALPHA * f2c)
        ones = jnp.ones((N, 1), jnp.float32)
        who = jnp.concatenate([wh, ones], axis=1)
        q = jnp.concatenate([who * g, who * gh], axis=1)
        tn = jnp.sum(q[:, F + 1:], axis=0, keepdims=True)
        for j in range(N // ROW_BLK):
            sl = slice(j * ROW_BLK, (j + 1) * ROW_BLK)
            mask = jnp.where(f2r >= -f1[sl, :], 1.0, 0.0)
            m = jnp.dot(mask, q, preferred_element_type=jnp.float32)
            mp = m[:, :F + 1]
            mn = tn - m[:, F + 1:]
            num = u[sl, :] * mp[:, :F] + v[sl, :] * mn[:, :F]
            den = u[sl, :] * mp[:, F:] + v[sl, :] * mn[:, F:]
            h = num / den
            write_out(j, jnp.where(h > 0, h, jnp.exp(h) - 1.0))

    w1 = w1_ref[...]
    a1 = a1_ref[...]
    w2 = w2_ref[...]
    a2 = a2_ref[...]
    for b in range(8):
        def write_h(j, val):
            h_s[pl.ds(j * ROW_BLK, ROW_BLK), :] = val

        def write_o(j, val, b=b):
            o_ref[b, pl.ds(j * ROW_BLK, ROW_BLK), :] = val

        layer(x_ref[b], w1, a1, write_h)
        layer(h_s[...], w2, a2, write_o)


@jax.jit
def kernel(x, W1, a1, W2, a2):
    B = x.shape[0]
    return pl.pallas_call(
        _fused_gat_body,
        out_shape=jax.ShapeDtypeStruct((B, N, F), jnp.float32),
        scratch_shapes=[
            pltpu.VMEM((N, F), jnp.float32),
            pltpu.VMEM((N, F), jnp.float32),
        ],
    )(x, W1, a1, W2, a2)


# speedup vs baseline: 1.3465x; 1.0542x over previous
import functools

import jax
import jax.numpy as jnp
from jax import lax
from jax.experimental import pallas as pl
from jax.experimental.pallas import tpu as pltpu

ALPHA = 0.2
N = 2048
F = 16
ROW_BLK = 256


def _fused_gat_body(x_ref, w1_ref, a1_ref, w2_ref, a2_ref, o_ref, wh_s, h_s):
    def layer(xin, W, a, write_out):
        wh = jnp.dot(xin, W, preferred_element_type=jnp.float32)
        wh_s[...] = wh
        f1 = jnp.dot(wh, a[:F, :], preferred_element_type=jnp.float32)
        f2c = jnp.dot(wh, a[F:, :], preferred_element_type=jnp.float32)
        f2r = lax.dot_general(
            a[F:, :], wh,
            dimension_numbers=(((0,), (1,)), ((), ())),
            preferred_element_type=jnp.float32,
        )
        u = jnp.exp(f1)
        v = jnp.exp(ALPHA * f1)
        g = jnp.exp(f2c)
        gh = jnp.exp(ALPHA * f2c)
        ones = jnp.ones((N, 1), jnp.float32)
        who = jnp.concatenate([wh, ones], axis=1)
        qf = jnp.concatenate([who * g, who * gh], axis=1)
        q = qf.astype(jnp.bfloat16)
        tn = jnp.sum(qf[:, F + 1:], axis=0, keepdims=True)
        f2rb = f2r.astype(jnp.bfloat16)
        nf1b = (-f1).astype(jnp.bfloat16)
        one_b = jnp.bfloat16(1.0)
        zero_b = jnp.bfloat16(0.0)
        for j in range(N // ROW_BLK):
            sl = slice(j * ROW_BLK, (j + 1) * ROW_BLK)
            mask = jnp.where(f2rb >= nf1b[sl, :], one_b, zero_b)
            m = jnp.dot(mask, q, preferred_element_type=jnp.float32)
            mp = m[:, :F + 1]
            mn = tn - m[:, F + 1:]
            num = u[sl, :] * mp[:, :F] + v[sl, :] * mn[:, :F]
            den = u[sl, :] * mp[:, F:] + v[sl, :] * mn[:, F:]
            h = num / den
            write_out(j, jnp.where(h > 0, h, jnp.exp(h) - 1.0))

    w1 = w1_ref[...]
    a1 = a1_ref[...]
    w2 = w2_ref[...]
    a2 = a2_ref[...]
    for b in range(8):
        def write_h(j, val):
            h_s[pl.ds(j * ROW_BLK, ROW_BLK), :] = val

        def write_o(j, val, b=b):
            o_ref[b, pl.ds(j * ROW_BLK, ROW_BLK), :] = val

        layer(x_ref[b], w1, a1, write_h)
        layer(h_s[...], w2, a2, write_o)


@jax.jit
def kernel(x, W1, a1, W2, a2):
    B = x.shape[0]
    return pl.pallas_call(
        _fused_gat_body,
        out_shape=jax.ShapeDtypeStruct((B, N, F), jnp.float32),
        scratch_shapes=[
            pltpu.VMEM((N, F), jnp.float32),
            pltpu.VMEM((N, F), jnp.float32),
        ],
    )(x, W1, a1, W2, a2)
